# baseline (speedup 1.0000x reference)
import functools

import jax
import jax.numpy as jnp
from jax import lax
from jax.experimental import pallas as pl
from jax.experimental.pallas import tpu as pltpu
from jax.experimental.pallas import tpu_sc as plsc

B = 16384
D = 32


@functools.lru_cache(maxsize=None)
def _build():
    info = plsc.get_sparse_core_info()
    NC, NS = info.num_cores, info.num_subcores
    NW = NC * NS
    b_per_w = B // NW

    mesh = plsc.VectorSubcoreMesh(core_axis_name="c", subcore_axis_name="s")
    out_sds = jax.ShapeDtypeStruct((B, D), jnp.float32)

    @functools.partial(
        pl.kernel,
        mesh=mesh,
        out_type=(out_sds, out_sds, out_sds),
        scratch_types=[
            pltpu.VMEM((b_per_w,), jnp.int32),
            pltpu.VMEM((b_per_w,), jnp.int32),
            pltpu.VMEM((b_per_w,), jnp.int32),
            pltpu.VMEM((b_per_w, D), jnp.float32),
            pltpu.VMEM((b_per_w, D), jnp.float32),
            pltpu.VMEM((b_per_w, D), jnp.float32),
            pltpu.SemaphoreType.DMA,
            pltpu.SemaphoreType.DMA,
            pltpu.SemaphoreType.DMA,
        ],
        compiler_params=pltpu.CompilerParams(use_tc_tiling_on_sc=False),
    )
    def body(uid, iid, cid, wu, wi, wc, ou, oi, oc,
             idx_u, idx_i, idx_c, rows_u, rows_i, rows_c,
             sem_idx, sem_g, sem_o):
        wid = lax.axis_index("s") * NC + lax.axis_index("c")
        base = wid * b_per_w
        sl = pl.ds(base, b_per_w)

        loads = [
            pltpu.async_copy(uid.at[sl], idx_u, sem_idx),
            pltpu.async_copy(iid.at[sl], idx_i, sem_idx),
            pltpu.async_copy(cid.at[sl], idx_c, sem_idx),
        ]
        for cp in loads:
            cp.wait()

        gathers = [
            pltpu.async_copy(wu.at[idx_u], rows_u, sem_g),
            pltpu.async_copy(wi.at[idx_i], rows_i, sem_g),
            pltpu.async_copy(wc.at[idx_c], rows_c, sem_g),
        ]
        for cp in gathers:
            cp.wait()

        stores = [
            pltpu.async_copy(rows_u, ou.at[sl], sem_o),
            pltpu.async_copy(rows_i, oi.at[sl], sem_o),
            pltpu.async_copy(rows_c, oc.at[sl], sem_o),
        ]
        for cp in stores:
            cp.wait()

    return body


def kernel(user_id, item_id, cate_id, W_user, W_item, W_cate):
    f = _build()
    return f(user_id.astype(jnp.int32), item_id.astype(jnp.int32),
             cate_id.astype(jnp.int32), W_user, W_item, W_cate)
